# Initial kernel scaffold; baseline (speedup 1.0000x reference)
import jax
import jax.numpy as jnp
from jax.experimental import pallas as pl


def kernel(x, emb, W1, b1, W2, b2):
    raise NotImplementedError("write your pallas kernel here")


# speedup vs baseline: 83.1024x; 83.1024x over previous
import functools

import jax
import jax.numpy as jnp
from jax import lax
from jax.experimental import pallas as pl
from jax.experimental.pallas import tpu as pltpu
from jax.experimental.pallas import tpu_sc as plsc

B = 16384
L = 200
V = 26
VP = 32
NC, NS, LANES = 2, 16, 16
NW = NC * NS
SPW = B // NW
CHUNK = 128
NCHUNK = SPW // CHUNK
GROUPS = CHUNK // LANES


def _sc_hist_kernel(x_hbm, out_hbm, x_buf, counts_buf):
  wid = lax.axis_index("s") * NC + lax.axis_index("c")
  wb = wid * SPW

  iota = lax.broadcasted_iota(jnp.int32, (LANES,), 0)
  ones = jnp.full((LANES,), 1.0, dtype=jnp.float32)
  zeros = jnp.zeros((LANES,), dtype=jnp.float32)

  def zero_body(i, _):
    counts_buf[pl.ds(i * LANES, LANES)] = zeros
    return 0
  lax.fori_loop(0, SPW * VP // LANES, zero_body, 0, unroll=8)

  for c in range(NCHUNK):
    pltpu.sync_copy(
        x_hbm.at[pl.ds((wb + c * CHUNK) * L, CHUNK * L)], x_buf)

    def group_body(g, _):
      row_off = (g * LANES + iota) * L
      dst_off = (c * CHUNK + g * LANES + iota) * VP

      def l_body(l, _):
        xv = plsc.load_gather(x_buf, [row_off + l])
        plsc.addupdate_scatter(counts_buf, [dst_off + xv], ones)
        return 0
      lax.fori_loop(0, L, l_body, 0, unroll=8)
      return 0
    lax.fori_loop(0, GROUPS, group_body, 0)

  pltpu.sync_copy(counts_buf, out_hbm.at[pl.ds(wb * VP, SPW * VP)])


@jax.jit
def _sc_hist(x_flat):
  mesh = plsc.VectorSubcoreMesh(core_axis_name="c", subcore_axis_name="s")
  fn = functools.partial(
      pl.kernel,
      mesh=mesh,
      compiler_params=pltpu.CompilerParams(needs_layout_passes=False),
      out_type=jax.ShapeDtypeStruct((B * VP,), jnp.float32),
      scratch_types=[
          pltpu.VMEM((CHUNK * L,), jnp.int32),
          pltpu.VMEM((SPW * VP,), jnp.float32),
      ],
  )(_sc_hist_kernel)
  return fn(x_flat)


BLK = 2048
OP = 8


def _tc_mlp_kernel(counts_ref, emb_ref, w1_ref, b1_ref, w2_ref, b2_ref,
                   out_ref):
  hi = lax.Precision.HIGHEST
  a = jnp.dot(emb_ref[...], w1_ref[...], precision=hi,
              preferred_element_type=jnp.float32)
  h = jnp.dot(counts_ref[...], a, precision=hi,
              preferred_element_type=jnp.float32)
  h = jnp.maximum(h + b1_ref[...], 0.0)
  out_ref[...] = (
      jnp.dot(h, w2_ref[...], precision=hi,
              preferred_element_type=jnp.float32)
      + b2_ref[...])


@jax.jit
def _tc_mlp(counts2d, emb_pad, W1, b1r, W2p, b2p):
  h = W1.shape[1]
  return pl.pallas_call(
      _tc_mlp_kernel,
      grid=(B // BLK,),
      in_specs=[
          pl.BlockSpec((BLK, VP), lambda i: (i, 0)),
          pl.BlockSpec((VP, emb_pad.shape[1]), lambda i: (0, 0)),
          pl.BlockSpec((emb_pad.shape[1], h), lambda i: (0, 0)),
          pl.BlockSpec((1, h), lambda i: (0, 0)),
          pl.BlockSpec((h, OP), lambda i: (0, 0)),
          pl.BlockSpec((1, OP), lambda i: (0, 0)),
      ],
      out_specs=pl.BlockSpec((BLK, OP), lambda i: (i, 0)),
      out_shape=jax.ShapeDtypeStruct((B, OP), jnp.float32),
  )(counts2d, emb_pad, W1, b1r, W2p, b2p)


def kernel(x, emb, W1, b1, W2, b2):
  x_flat = x.astype(jnp.int32).reshape(-1)
  counts = _sc_hist(x_flat).reshape(B, VP)

  d = emb.shape[1]
  o = W2.shape[1]
  emb_pad = jnp.zeros((VP, d), jnp.float32).at[:V].set(emb)
  W2p = jnp.zeros((W2.shape[0], OP), jnp.float32).at[:, :o].set(W2)
  b2p = jnp.zeros((1, OP), jnp.float32).at[:, :o].set(b2)

  out = _tc_mlp(counts, emb_pad, W1, b1.reshape(1, -1), W2p, b2p)
  return out[:, :o]


# speedup vs baseline: 120.6415x; 1.4517x over previous
import functools

import jax
import jax.numpy as jnp
from jax import lax
from jax.experimental import pallas as pl
from jax.experimental.pallas import tpu as pltpu
from jax.experimental.pallas import tpu_sc as plsc

B = 16384
L = 200
V = 26
VP = 32
NC, NS, LANES = 2, 16, 16
NW = NC * NS
SPW = B // NW
CHUNK = 128
NCHUNK = SPW // CHUNK
GROUPS = CHUNK // LANES


def _sc_hist_kernel(x_hbm, out_hbm, x_buf0, x_buf1, counts_buf, sem0, sem1):
  wid = lax.axis_index("s") * NC + lax.axis_index("c")
  wb = wid * SPW

  iota = lax.broadcasted_iota(jnp.int32, (LANES,), 0)
  ones = jnp.full((LANES,), 1.0, dtype=jnp.float32)
  zeros = jnp.zeros((LANES,), dtype=jnp.float32)
  sems = (sem0, sem1)
  bufs = (x_buf0, x_buf1)

  def start(c):
    return pltpu.async_copy(
        x_hbm.at[pl.ds((wb + c * CHUNK) * L, CHUNK * L)],
        bufs[c % 2], sems[c % 2])

  cp = start(0)

  @plsc.parallel_loop(0, SPW * VP // LANES, unroll=8)
  def _(i):
    counts_buf[pl.ds(i * LANES, LANES)] = zeros

  for c in range(NCHUNK):
    nxt = start(c + 1) if c + 1 < NCHUNK else None
    cp.wait()
    buf = bufs[c % 2]

    def group_body(g, _):
      row_off = (g * LANES + iota) * L
      dst_off = (c * CHUNK + g * LANES + iota) * VP

      @plsc.parallel_loop(0, L, unroll=8)
      def _(l):
        xv = plsc.load_gather(buf, [row_off + l])
        plsc.addupdate_scatter(counts_buf, [dst_off + xv], ones)
      return 0
    lax.fori_loop(0, GROUPS, group_body, 0)
    cp = nxt

  pltpu.sync_copy(counts_buf, out_hbm.at[pl.ds(wb * VP, SPW * VP)])


@jax.jit
def _sc_hist(x_flat):
  mesh = plsc.VectorSubcoreMesh(core_axis_name="c", subcore_axis_name="s")
  fn = functools.partial(
      pl.kernel,
      mesh=mesh,
      compiler_params=pltpu.CompilerParams(needs_layout_passes=False),
      out_type=jax.ShapeDtypeStruct((B * VP,), jnp.float32),
      scratch_types=[
          pltpu.VMEM((CHUNK * L,), jnp.int32),
          pltpu.VMEM((CHUNK * L,), jnp.int32),
          pltpu.VMEM((SPW * VP,), jnp.float32),
          pltpu.SemaphoreType.DMA,
          pltpu.SemaphoreType.DMA,
      ],
  )(_sc_hist_kernel)
  return fn(x_flat)


BLK = 2048
OP = 8


def _tc_mlp_kernel(counts_ref, emb_ref, w1_ref, b1_ref, w2_ref, b2_ref,
                   out_ref):
  hi = lax.Precision.HIGHEST
  a = jnp.dot(emb_ref[...], w1_ref[...], precision=hi,
              preferred_element_type=jnp.float32)
  h = jnp.dot(counts_ref[...], a, precision=hi,
              preferred_element_type=jnp.float32)
  h = jnp.maximum(h + b1_ref[...], 0.0)
  out_ref[...] = (
      jnp.dot(h, w2_ref[...], precision=hi,
              preferred_element_type=jnp.float32)
      + b2_ref[...])


@jax.jit
def _tc_mlp(counts2d, emb_pad, W1, b1r, W2p, b2p):
  h = W1.shape[1]
  return pl.pallas_call(
      _tc_mlp_kernel,
      grid=(B // BLK,),
      in_specs=[
          pl.BlockSpec((BLK, VP), lambda i: (i, 0)),
          pl.BlockSpec((VP, emb_pad.shape[1]), lambda i: (0, 0)),
          pl.BlockSpec((emb_pad.shape[1], h), lambda i: (0, 0)),
          pl.BlockSpec((1, h), lambda i: (0, 0)),
          pl.BlockSpec((h, OP), lambda i: (0, 0)),
          pl.BlockSpec((1, OP), lambda i: (0, 0)),
      ],
      out_specs=pl.BlockSpec((BLK, OP), lambda i: (i, 0)),
      out_shape=jax.ShapeDtypeStruct((B, OP), jnp.float32),
  )(counts2d, emb_pad, W1, b1r, W2p, b2p)


def kernel(x, emb, W1, b1, W2, b2):
  x_flat = x.astype(jnp.int32).reshape(-1)
  counts = _sc_hist(x_flat).reshape(B, VP)

  d = emb.shape[1]
  o = W2.shape[1]
  emb_pad = jnp.zeros((VP, d), jnp.float32).at[:V].set(emb)
  W2p = jnp.zeros((W2.shape[0], OP), jnp.float32).at[:, :o].set(W2)
  b2p = jnp.zeros((1, OP), jnp.float32).at[:, :o].set(b2)

  out = _tc_mlp(counts, emb_pad, W1, b1.reshape(1, -1), W2p, b2p)
  return out[:, :o]
